# Initial kernel scaffold; baseline (speedup 1.0000x reference)
import jax
import jax.numpy as jnp
from jax.experimental import pallas as pl


def kernel(feat_src, feat_dst, edge_index, W, b, prelu_a, fc_W, fc_b, attn):
    raise NotImplementedError("write your pallas kernel here")


# speedup vs baseline: 4.9397x; 4.9397x over previous
import functools

import jax
import jax.numpy as jnp
from jax import lax
from jax.experimental import pallas as pl
from jax.experimental.pallas import tpu as pltpu
from jax.experimental.pallas import tpu_sc as plsc

_N_S = 5000
_N_D = 5000
_E = 320000
_D = 128

_B = 128
_RP = 5120
_RPT = _RP // 16
_SS = 16
_CP = 160
_SP = _CP // _SS
_CN = 80
_SN = _CN // _SS
_TRASH = _RP - 1

_COPY_PIECES = ((0, 128), (128, 128), (256, _RPT - 256))


def _prelu(x, a):
    return jnp.where(x > 0, x, a * x)


def _tc_norm_body(x_ref, w_ref, b_ref, a_ref, t_ref):
    x = x_ref[...]
    h = jnp.dot(x, w_ref[...], preferred_element_type=jnp.float32)
    h = _prelu(h + b_ref[...], a_ref[0, 0])
    n2 = jnp.sum(h * h, axis=1, keepdims=True)
    t_ref[:_N_S, :] = x
    t_ref[_N_S:, :] = h * lax.rsqrt(jnp.maximum(n2, 1e-16))


def _tc_norm(x, w, b2, a2):
    return pl.pallas_call(
        _tc_norm_body,
        out_shape=jax.ShapeDtypeStruct((2 * _N_S, _D), jnp.float32),
    )(x, w, b2, a2)


_sc_mesh = plsc.VectorSubcoreMesh(core_axis_name="c", subcore_axis_name="s")


@functools.partial(
    pl.kernel,
    out_type=(
        jax.ShapeDtypeStruct((2, _RP, _D), jnp.float32),
        jax.ShapeDtypeStruct((2, _RP, _D), jnp.float32),
        jax.ShapeDtypeStruct((_RP,), jnp.float32),
        jax.ShapeDtypeStruct((_RP,), jnp.float32),
    ),
    mesh=_sc_mesh,
    scratch_types=[
        pltpu.VMEM_SHARED((_RP, _D), jnp.float32),
        pltpu.VMEM_SHARED((_RP, _D), jnp.float32),
        pltpu.VMEM_SHARED((_RP,), jnp.float32),
        pltpu.VMEM_SHARED((_RP,), jnp.float32),
        pltpu.VMEM((_SS, _B), jnp.int32),
        pltpu.VMEM((_SS, _B), jnp.int32),
        pltpu.VMEM((_B, _D), jnp.float32),
        pltpu.VMEM((_B,), jnp.float32),
        pltpu.SemaphoreType.DMA,
    ],
)
def _sc_segsum(srcp, dstp, srcn, dstn, table,
               z128, z1, ones_h,
               sump_o, sumn_o, dp_o, dn_o,
               acc0, acc1, dacc0, dacc1, idxs, idxd, rows, ones_v, sem):
    c = lax.axis_index("c")
    s = lax.axis_index("s")
    r0 = s * _RPT

    pltpu.sync_copy(z128, rows)
    pltpu.sync_copy(z1, ones_v)
    for off, sz in _COPY_PIECES:
        pltpu.sync_copy(rows.at[pl.ds(0, sz)], acc0.at[pl.ds(r0 + off, sz)])
        pltpu.sync_copy(rows.at[pl.ds(0, sz)], acc1.at[pl.ds(r0 + off, sz)])
        pltpu.sync_copy(ones_v.at[pl.ds(0, sz)], dacc0.at[pl.ds(r0 + off, sz)])
        pltpu.sync_copy(ones_v.at[pl.ds(0, sz)], dacc1.at[pl.ds(r0 + off, sz)])
    pltpu.sync_copy(ones_h, ones_v)
    plsc.subcore_barrier()

    def run_pass(src_h, dst_h, nstages, accum, daccum):
        def stage(t, carry):
            pltpu.sync_copy(src_h.at[c, s, pl.ds(t * _SS, _SS)], idxs)
            pltpu.sync_copy(dst_h.at[s, pl.ds(t * _SS, _SS)], idxd)

            def chunk(j, cc):
                pltpu.async_copy(table.at[idxs.at[j]], rows, sem).wait()
                pltpu.sync_copy(rows, accum.at[idxd.at[j]], add=True)

                @pl.when(c == 0)
                def _():
                    pltpu.sync_copy(ones_v, daccum.at[idxd.at[j]], add=True)

                return cc

            lax.fori_loop(0, _SS, chunk, 0)
            return carry

        lax.fori_loop(0, nstages, stage, 0)

    run_pass(srcp, dstp, _SP, acc0, dacc0)
    run_pass(srcn, dstn, _SN, acc1, dacc1)
    plsc.subcore_barrier()

    def copy_out(accum, out):
        for off, sz in _COPY_PIECES:
            pltpu.sync_copy(accum.at[pl.ds(r0 + off, sz)],
                            rows.at[pl.ds(0, sz)])
            pltpu.sync_copy(rows.at[pl.ds(0, sz)],
                            out.at[c, pl.ds(r0 + off, sz)])

    copy_out(acc0, sump_o)
    copy_out(acc1, sumn_o)

    @pl.when(c == 0)
    def _():
        for daccum, out in ((dacc0, dp_o), (dacc1, dn_o)):
            for off, sz in _COPY_PIECES:
                pltpu.sync_copy(daccum.at[pl.ds(r0 + off, sz)],
                                ones_v.at[pl.ds(0, sz)])
                pltpu.sync_copy(ones_v.at[pl.ds(0, sz)],
                                out.at[pl.ds(r0 + off, sz)])


def _tc_final_body(ap_ref, an_ref, gp_ref, gn_ref, dp_ref, dn_ref, fd_ref,
                   w_ref, b_ref, a_ref, z_ref, loss_ref):
    fd = fd_ref[...]
    wm = w_ref[...]
    bb = b_ref[...]
    aa = a_ref[0, 0]
    mp = (ap_ref[...] + fd) / (dp_ref[...] + 1.0)
    hp = _prelu(jnp.dot(mp, wm, preferred_element_type=jnp.float32) + bb, aa)
    mn = (an_ref[...] + fd) / (dn_ref[...] + 1.0)
    hn = _prelu(jnp.dot(mn, wm, preferred_element_type=jnp.float32) + bb, aa)
    npn = jnp.sqrt(jnp.sum(hp * hp, axis=1, keepdims=True))
    nnn = jnp.sqrt(jnp.sum(hn * hn, axis=1, keepdims=True))
    pos = jnp.sum(hp * hn, axis=1, keepdims=True) / jnp.maximum(npn * nnn, 1e-8)
    neg1 = jnp.sum(hp * gn_ref[...], axis=1, keepdims=True) \
        / jnp.maximum(npn, 1e-20) + pos
    neg2 = jnp.sum(hn * gp_ref[...], axis=1, keepdims=True) \
        / jnp.maximum(nnn, 1e-20) + pos
    stot = jnp.sum(jnp.exp(pos)) + jnp.sum(jnp.exp(neg1)) + jnp.sum(jnp.exp(neg2))
    loss_ref[...] = (jnp.log(stot) - jnp.sum(pos)).reshape(1, 1)
    z_ref[...] = hp


def _tc_final(ap, an, gp, gn, dp, dn, fd, w, b2, a2):
    return pl.pallas_call(
        _tc_final_body,
        out_shape=(
            jax.ShapeDtypeStruct((_N_D, _D), jnp.float32),
            jax.ShapeDtypeStruct((1, 1), jnp.float32),
        ),
    )(ap, an, gp, gn, dp, dn, fd, w, b2, a2)


def kernel(feat_src, feat_dst, edge_index, W, b, prelu_a, fc_W, fc_b, attn):
    del fc_W, fc_b, attn
    src = edge_index[0]
    dst = edge_index[1]
    b2 = b.reshape(1, _D)
    a2 = prelu_a.reshape(1, 1)

    table = _tc_norm(feat_src, W, b2, a2)

    pad_p = 16 * _CP * _B - _E
    pad_n = 16 * _CN * _B - _E // 2
    i32 = src.dtype
    srcp = jnp.concatenate([src, jnp.zeros((pad_p,), i32)]).reshape(16, _CP, _B)
    dstp = jnp.concatenate([dst, jnp.full((pad_p,), _TRASH, i32)]).reshape(16, _CP, _B)
    se = src[::2]
    de = dst[::2]
    srcn = jnp.concatenate([se, jnp.zeros((pad_n,), i32)]).reshape(16, _CN, _B)
    dstn = jnp.concatenate([de, jnp.full((pad_n,), _TRASH, i32)]).reshape(16, _CN, _B)
    srcp2 = jnp.stack([srcp, srcp + _N_S])
    srcn2 = jnp.stack([srcn, srcn + _N_S])

    z128 = jnp.zeros((_B, _D), jnp.float32)
    z1 = jnp.zeros((_B,), jnp.float32)
    ones = jnp.ones((_B,), jnp.float32)

    sump, sumn, dp, dn = _sc_segsum(
        srcp2, dstp, srcn2, dstn, table, z128, z1, ones)

    z, loss = _tc_final(
        sump[0, :_N_D], sumn[0, :_N_D], sump[1, :_N_D], sumn[1, :_N_D],
        dp[:_N_D, None], dn[:_N_D, None], feat_dst, W, b2, a2)
    return loss[0, 0], z


# speedup vs baseline: 5.7090x; 1.1557x over previous
import functools

import jax
import jax.numpy as jnp
from jax import lax
from jax.experimental import pallas as pl
from jax.experimental.pallas import tpu as pltpu
from jax.experimental.pallas import tpu_sc as plsc

_N_S = 5000
_N_D = 5000
_E = 320000
_D = 128

_B = 128
_RP = 5120
_RPT = _RP // 16
_SS = 16
_CP = 160
_SP = _CP // _SS
_CN = 80
_SN = _CN // _SS
_TRASH = _RP - 1

_COPY_PIECES = ((0, 128), (128, 128), (256, _RPT - 256))


def _prelu(x, a):
    return jnp.where(x > 0, x, a * x)


def _tc_norm_body(x_ref, w_ref, b_ref, a_ref, t_ref):
    x = x_ref[...]
    h = jnp.dot(x, w_ref[...], preferred_element_type=jnp.float32)
    h = _prelu(h + b_ref[...], a_ref[0, 0])
    n2 = jnp.sum(h * h, axis=1, keepdims=True)
    t_ref[:_N_S, :] = x
    t_ref[_N_S:, :] = h * lax.rsqrt(jnp.maximum(n2, 1e-16))


def _tc_norm(x, w, b2, a2):
    return pl.pallas_call(
        _tc_norm_body,
        out_shape=jax.ShapeDtypeStruct((2 * _N_S, _D), jnp.float32),
    )(x, w, b2, a2)


_sc_mesh = plsc.VectorSubcoreMesh(core_axis_name="c", subcore_axis_name="s")


@functools.partial(
    pl.kernel,
    out_type=(
        jax.ShapeDtypeStruct((2, _RP, _D), jnp.float32),
        jax.ShapeDtypeStruct((2, _RP, _D), jnp.float32),
        jax.ShapeDtypeStruct((_RP,), jnp.float32),
        jax.ShapeDtypeStruct((_RP,), jnp.float32),
    ),
    mesh=_sc_mesh,
    scratch_types=[
        pltpu.VMEM_SHARED((_RP, _D), jnp.float32),
        pltpu.VMEM_SHARED((_RP, _D), jnp.float32),
        pltpu.VMEM_SHARED((_RP,), jnp.float32),
        pltpu.VMEM_SHARED((_RP,), jnp.float32),
        pltpu.VMEM((2, _SS, _B), jnp.int32),
        pltpu.VMEM((2, _SS, _B), jnp.int32),
        pltpu.VMEM((2, _B, _D), jnp.float32),
        pltpu.VMEM((_B,), jnp.float32),
        pltpu.SemaphoreType.DMA,
        pltpu.SemaphoreType.DMA,
        pltpu.SemaphoreType.DMA,
        pltpu.SemaphoreType.DMA,
    ],
)
def _sc_segsum(srcp, dstp, srcn, dstn, table,
               z128, z1, ones_h,
               sump_o, sumn_o, dp_o, dn_o,
               acc0, acc1, dacc0, dacc1, idxs, idxd, rows, ones_v,
               sem_g, sem_s, sem_d, sem_i):
    c = lax.axis_index("c")
    s = lax.axis_index("s")
    r0 = s * _RPT

    pltpu.sync_copy(z128, rows.at[0])
    pltpu.sync_copy(z1, ones_v)
    for off, sz in _COPY_PIECES:
        pltpu.sync_copy(rows.at[0, pl.ds(0, sz)], acc0.at[pl.ds(r0 + off, sz)])
        pltpu.sync_copy(rows.at[0, pl.ds(0, sz)], acc1.at[pl.ds(r0 + off, sz)])
        pltpu.sync_copy(ones_v.at[pl.ds(0, sz)], dacc0.at[pl.ds(r0 + off, sz)])
        pltpu.sync_copy(ones_v.at[pl.ds(0, sz)], dacc1.at[pl.ds(r0 + off, sz)])
    pltpu.sync_copy(ones_h, ones_v)
    plsc.subcore_barrier()

    def run_pass(src_h, dst_h, nstages, accum, daccum):
        pltpu.sync_copy(src_h.at[c, s, pl.ds(0, _SS)], idxs.at[0])
        pltpu.sync_copy(dst_h.at[s, pl.ds(0, _SS)], idxd.at[0])

        def stage(t, carry):
            tb = lax.rem(t, 2)
            nb = lax.rem(t + 1, 2)

            @pl.when(t + 1 < nstages)
            def _():
                pltpu.async_copy(
                    src_h.at[c, s, pl.ds((t + 1) * _SS, _SS)], idxs.at[nb], sem_i)
                pltpu.async_copy(
                    dst_h.at[s, pl.ds((t + 1) * _SS, _SS)], idxd.at[nb], sem_i)

            pltpu.async_copy(table.at[idxs.at[tb, 0]], rows.at[0], sem_g)

            def chunk(j, cc):
                pb = lax.rem(j, 2)
                qb = lax.rem(j + 1, 2)
                pltpu.make_async_copy(
                    table.at[idxs.at[tb, j]], rows.at[pb], sem_g).wait()
                pltpu.async_copy(
                    rows.at[pb], accum.at[idxd.at[tb, j]], sem_s, add=True)

                @pl.when(c == 0)
                def _():
                    pltpu.async_copy(
                        ones_v, daccum.at[idxd.at[tb, j]], sem_d, add=True)

                @pl.when(j > 0)
                def _():
                    pltpu.make_async_copy(
                        z128, accum.at[pl.ds(0, _B)], sem_s).wait()

                @pl.when(jnp.logical_and(c == 0, j > 0))
                def _():
                    pltpu.make_async_copy(
                        z1, daccum.at[pl.ds(0, _B)], sem_d).wait()

                @pl.when(j + 1 < _SS)
                def _():
                    pltpu.async_copy(
                        table.at[idxs.at[tb, j + 1]], rows.at[qb], sem_g)

                return cc

            lax.fori_loop(0, _SS, chunk, 0)
            pltpu.make_async_copy(z128, accum.at[pl.ds(0, _B)], sem_s).wait()

            @pl.when(c == 0)
            def _():
                pltpu.make_async_copy(z1, daccum.at[pl.ds(0, _B)], sem_d).wait()

            @pl.when(t + 1 < nstages)
            def _():
                pltpu.make_async_copy(
                    src_h.at[c, s, pl.ds((t + 1) * _SS, _SS)],
                    idxs.at[nb], sem_i).wait()
                pltpu.make_async_copy(
                    dst_h.at[s, pl.ds((t + 1) * _SS, _SS)],
                    idxd.at[nb], sem_i).wait()

            return carry

        lax.fori_loop(0, nstages, stage, 0)

    run_pass(srcp, dstp, _SP, acc0, dacc0)
    run_pass(srcn, dstn, _SN, acc1, dacc1)
    plsc.subcore_barrier()

    def copy_out(accum, out):
        for off, sz in _COPY_PIECES:
            pltpu.sync_copy(accum.at[pl.ds(r0 + off, sz)],
                            rows.at[0, pl.ds(0, sz)])
            pltpu.sync_copy(rows.at[0, pl.ds(0, sz)],
                            out.at[c, pl.ds(r0 + off, sz)])

    copy_out(acc0, sump_o)
    copy_out(acc1, sumn_o)

    @pl.when(c == 0)
    def _():
        for daccum, out in ((dacc0, dp_o), (dacc1, dn_o)):
            for off, sz in _COPY_PIECES:
                pltpu.sync_copy(daccum.at[pl.ds(r0 + off, sz)],
                                ones_v.at[pl.ds(0, sz)])
                pltpu.sync_copy(ones_v.at[pl.ds(0, sz)],
                                out.at[pl.ds(r0 + off, sz)])


def _tc_final_body(ap_ref, an_ref, gp_ref, gn_ref, dp_ref, dn_ref, fd_ref,
                   w_ref, b_ref, a_ref, z_ref, loss_ref):
    fd = fd_ref[...]
    wm = w_ref[...]
    bb = b_ref[...]
    aa = a_ref[0, 0]
    mp = (ap_ref[...] + fd) / (dp_ref[...] + 1.0)
    hp = _prelu(jnp.dot(mp, wm, preferred_element_type=jnp.float32) + bb, aa)
    mn = (an_ref[...] + fd) / (dn_ref[...] + 1.0)
    hn = _prelu(jnp.dot(mn, wm, preferred_element_type=jnp.float32) + bb, aa)
    npn = jnp.sqrt(jnp.sum(hp * hp, axis=1, keepdims=True))
    nnn = jnp.sqrt(jnp.sum(hn * hn, axis=1, keepdims=True))
    pos = jnp.sum(hp * hn, axis=1, keepdims=True) / jnp.maximum(npn * nnn, 1e-8)
    neg1 = jnp.sum(hp * gn_ref[...], axis=1, keepdims=True) \
        / jnp.maximum(npn, 1e-20) + pos
    neg2 = jnp.sum(hn * gp_ref[...], axis=1, keepdims=True) \
        / jnp.maximum(nnn, 1e-20) + pos
    stot = jnp.sum(jnp.exp(pos)) + jnp.sum(jnp.exp(neg1)) + jnp.sum(jnp.exp(neg2))
    loss_ref[...] = (jnp.log(stot) - jnp.sum(pos)).reshape(1, 1)
    z_ref[...] = hp


def _tc_final(ap, an, gp, gn, dp, dn, fd, w, b2, a2):
    return pl.pallas_call(
        _tc_final_body,
        out_shape=(
            jax.ShapeDtypeStruct((_N_D, _D), jnp.float32),
            jax.ShapeDtypeStruct((1, 1), jnp.float32),
        ),
    )(ap, an, gp, gn, dp, dn, fd, w, b2, a2)


def kernel(feat_src, feat_dst, edge_index, W, b, prelu_a, fc_W, fc_b, attn):
    del fc_W, fc_b, attn
    src = edge_index[0]
    dst = edge_index[1]
    b2 = b.reshape(1, _D)
    a2 = prelu_a.reshape(1, 1)

    table = _tc_norm(feat_src, W, b2, a2)

    pad_p = 16 * _CP * _B - _E
    pad_n = 16 * _CN * _B - _E // 2
    i32 = src.dtype
    srcp = jnp.concatenate([src, jnp.zeros((pad_p,), i32)]).reshape(16, _CP, _B)
    dstp = jnp.concatenate([dst, jnp.full((pad_p,), _TRASH, i32)]).reshape(16, _CP, _B)
    se = src[::2]
    de = dst[::2]
    srcn = jnp.concatenate([se, jnp.zeros((pad_n,), i32)]).reshape(16, _CN, _B)
    dstn = jnp.concatenate([de, jnp.full((pad_n,), _TRASH, i32)]).reshape(16, _CN, _B)
    srcp2 = jnp.stack([srcp, srcp + _N_S])
    srcn2 = jnp.stack([srcn, srcn + _N_S])

    z128 = jnp.zeros((_B, _D), jnp.float32)
    z1 = jnp.zeros((_B,), jnp.float32)
    ones = jnp.ones((_B,), jnp.float32)

    sump, sumn, dp, dn = _sc_segsum(
        srcp2, dstp, srcn2, dstn, table, z128, z1, ones)

    z, loss = _tc_final(
        sump[0, :_N_D], sumn[0, :_N_D], sump[1, :_N_D], sumn[1, :_N_D],
        dp[:_N_D, None], dn[:_N_D, None], feat_dst, W, b2, a2)
    return loss[0, 0], z


# speedup vs baseline: 7.9432x; 1.3913x over previous
import functools

import jax
import jax.numpy as jnp
from jax import lax
from jax.experimental import pallas as pl
from jax.experimental.pallas import tpu as pltpu
from jax.experimental.pallas import tpu_sc as plsc

_N_S = 5000
_N_D = 5000
_E = 320000
_D = 128

_B = 128
_RP = 5120
_RPT = _RP // 16
_SS = 16
_CP = 160
_SP = _CP // _SS
_CN = 80
_SN = _CN // _SS
_TRASH = _RP - 1

_COPY_PIECES = ((0, 128), (128, 128), (256, _RPT - 256))


def _prelu(x, a):
    return jnp.where(x > 0, x, a * x)


def _tc_norm_body(x_ref, w_ref, b_ref, a_ref, t_ref):
    x = x_ref[...]
    h = jnp.dot(x, w_ref[...], preferred_element_type=jnp.float32)
    h = _prelu(h + b_ref[...], a_ref[0, 0])
    n2 = jnp.sum(h * h, axis=1, keepdims=True)
    t_ref[:_N_S, :] = x
    t_ref[_N_S:, :] = h * lax.rsqrt(jnp.maximum(n2, 1e-16))


def _tc_norm(x, w, b2, a2):
    return pl.pallas_call(
        _tc_norm_body,
        out_shape=jax.ShapeDtypeStruct((2 * _N_S, _D), jnp.float32),
    )(x, w, b2, a2)


_sc_mesh = plsc.VectorSubcoreMesh(core_axis_name="c", subcore_axis_name="s")


@functools.partial(
    pl.kernel,
    out_type=(
        jax.ShapeDtypeStruct((2, _RP, _D), jnp.float32),
        jax.ShapeDtypeStruct((2, _RP, _D), jnp.float32),
        jax.ShapeDtypeStruct((_RP,), jnp.float32),
        jax.ShapeDtypeStruct((_RP,), jnp.float32),
    ),
    mesh=_sc_mesh,
    scratch_types=[
        pltpu.VMEM_SHARED((_RP, _D), jnp.float32),
        pltpu.VMEM_SHARED((_RP, _D), jnp.float32),
        pltpu.VMEM_SHARED((_RP,), jnp.float32),
        pltpu.VMEM_SHARED((_RP,), jnp.float32),
        pltpu.VMEM((2, _SS, _B), jnp.int32),
        pltpu.VMEM((2, _SS, _B), jnp.int32),
        pltpu.VMEM((2, _SS, _B), jnp.int32),
        pltpu.VMEM((2, _B, _D), jnp.float32),
        pltpu.VMEM((_B,), jnp.float32),
        pltpu.SemaphoreType.DMA,
        pltpu.SemaphoreType.DMA,
        pltpu.SemaphoreType.DMA,
        pltpu.SemaphoreType.DMA,
    ],
)
def _sc_segsum(srcp, dstp, dstm, table,
               z128, z1, ones_h,
               sump_o, sumn_o, dp_o, dn_o,
               acc0, acc1, dacc0, dacc1, idxs, idxd, idxm, rows, ones_v,
               sem_g, sem_s, sem_d, sem_i):
    c = lax.axis_index("c")
    s = lax.axis_index("s")
    r0 = s * _RPT

    pltpu.sync_copy(z128, rows.at[0])
    pltpu.sync_copy(z1, ones_v)
    for off, sz in _COPY_PIECES:
        pltpu.sync_copy(rows.at[0, pl.ds(0, sz)], acc0.at[pl.ds(r0 + off, sz)])
        pltpu.sync_copy(rows.at[0, pl.ds(0, sz)], acc1.at[pl.ds(r0 + off, sz)])
        pltpu.sync_copy(ones_v.at[pl.ds(0, sz)], dacc0.at[pl.ds(r0 + off, sz)])
        pltpu.sync_copy(ones_v.at[pl.ds(0, sz)], dacc1.at[pl.ds(r0 + off, sz)])
    pltpu.sync_copy(ones_h, ones_v)
    plsc.subcore_barrier()

    def stage(t, carry):
        tb = lax.rem(t, 2)
        nb = lax.rem(t + 1, 2)

        @pl.when(t + 1 < _SP)
        def _():
            pltpu.async_copy(
                srcp.at[c, s, pl.ds((t + 1) * _SS, _SS)], idxs.at[nb], sem_i)
            pltpu.async_copy(
                dstp.at[s, pl.ds((t + 1) * _SS, _SS)], idxd.at[nb], sem_i)
            pltpu.async_copy(
                dstm.at[s, pl.ds((t + 1) * _SS, _SS)], idxm.at[nb], sem_i)

        pltpu.async_copy(table.at[idxs.at[tb, 0]], rows.at[0], sem_g)

        def chunk(j, cc):
            pb = lax.rem(j, 2)
            qb = lax.rem(j + 1, 2)
            pltpu.make_async_copy(
                table.at[idxs.at[tb, j]], rows.at[pb], sem_g).wait()
            pltpu.async_copy(
                rows.at[pb], acc0.at[idxd.at[tb, j]], sem_s, add=True)
            pltpu.async_copy(
                rows.at[pb], acc1.at[idxm.at[tb, j]], sem_s, add=True)

            @pl.when(c == 0)
            def _():
                pltpu.async_copy(
                    ones_v, dacc0.at[idxd.at[tb, j]], sem_d, add=True)
                pltpu.async_copy(
                    ones_v, dacc1.at[idxm.at[tb, j]], sem_d, add=True)

            @pl.when(j > 0)
            def _():
                pltpu.make_async_copy(z128, acc0.at[pl.ds(0, _B)], sem_s).wait()
                pltpu.make_async_copy(z128, acc1.at[pl.ds(0, _B)], sem_s).wait()

            @pl.when(jnp.logical_and(c == 0, j > 0))
            def _():
                pltpu.make_async_copy(z1, dacc0.at[pl.ds(0, _B)], sem_d).wait()
                pltpu.make_async_copy(z1, dacc1.at[pl.ds(0, _B)], sem_d).wait()

            @pl.when(j + 1 < _SS)
            def _():
                pltpu.async_copy(
                    table.at[idxs.at[tb, j + 1]], rows.at[qb], sem_g)

            return cc

        lax.fori_loop(0, _SS, chunk, 0)
        pltpu.make_async_copy(z128, acc0.at[pl.ds(0, _B)], sem_s).wait()
        pltpu.make_async_copy(z128, acc1.at[pl.ds(0, _B)], sem_s).wait()

        @pl.when(c == 0)
        def _():
            pltpu.make_async_copy(z1, dacc0.at[pl.ds(0, _B)], sem_d).wait()
            pltpu.make_async_copy(z1, dacc1.at[pl.ds(0, _B)], sem_d).wait()

        @pl.when(t + 1 < _SP)
        def _():
            pltpu.make_async_copy(
                srcp.at[c, s, pl.ds((t + 1) * _SS, _SS)], idxs.at[nb], sem_i).wait()
            pltpu.make_async_copy(
                dstp.at[s, pl.ds((t + 1) * _SS, _SS)], idxd.at[nb], sem_i).wait()
            pltpu.make_async_copy(
                dstm.at[s, pl.ds((t + 1) * _SS, _SS)], idxm.at[nb], sem_i).wait()

        return carry

    pltpu.sync_copy(srcp.at[c, s, pl.ds(0, _SS)], idxs.at[0])
    pltpu.sync_copy(dstp.at[s, pl.ds(0, _SS)], idxd.at[0])
    pltpu.sync_copy(dstm.at[s, pl.ds(0, _SS)], idxm.at[0])
    lax.fori_loop(0, _SP, stage, 0)
    plsc.subcore_barrier()

    def copy_out(accum, out):
        for off, sz in _COPY_PIECES:
            pltpu.sync_copy(accum.at[pl.ds(r0 + off, sz)],
                            rows.at[0, pl.ds(0, sz)])
            pltpu.sync_copy(rows.at[0, pl.ds(0, sz)],
                            out.at[c, pl.ds(r0 + off, sz)])

    copy_out(acc0, sump_o)
    copy_out(acc1, sumn_o)

    @pl.when(c == 0)
    def _():
        for daccum, out in ((dacc0, dp_o), (dacc1, dn_o)):
            for off, sz in _COPY_PIECES:
                pltpu.sync_copy(daccum.at[pl.ds(r0 + off, sz)],
                                ones_v.at[pl.ds(0, sz)])
                pltpu.sync_copy(ones_v.at[pl.ds(0, sz)],
                                out.at[pl.ds(r0 + off, sz)])


def _tc_final_body(ap_ref, an_ref, gp_ref, gn_ref, dp_ref, dn_ref, fd_ref,
                   w_ref, b_ref, a_ref, z_ref, loss_ref):
    fd = fd_ref[...]
    wm = w_ref[...]
    bb = b_ref[...]
    aa = a_ref[0, 0]
    mp = (ap_ref[...] + fd) / (dp_ref[...] + 1.0)
    hp = _prelu(jnp.dot(mp, wm, preferred_element_type=jnp.float32) + bb, aa)
    mn = (an_ref[...] + fd) / (dn_ref[...] + 1.0)
    hn = _prelu(jnp.dot(mn, wm, preferred_element_type=jnp.float32) + bb, aa)
    npn = jnp.sqrt(jnp.sum(hp * hp, axis=1, keepdims=True))
    nnn = jnp.sqrt(jnp.sum(hn * hn, axis=1, keepdims=True))
    pos = jnp.sum(hp * hn, axis=1, keepdims=True) / jnp.maximum(npn * nnn, 1e-8)
    neg1 = jnp.sum(hp * gn_ref[...], axis=1, keepdims=True) \
        / jnp.maximum(npn, 1e-20) + pos
    neg2 = jnp.sum(hn * gp_ref[...], axis=1, keepdims=True) \
        / jnp.maximum(nnn, 1e-20) + pos
    stot = jnp.sum(jnp.exp(pos)) + jnp.sum(jnp.exp(neg1)) + jnp.sum(jnp.exp(neg2))
    loss_ref[...] = (jnp.log(stot) - jnp.sum(pos)).reshape(1, 1)
    z_ref[...] = hp


def _tc_final(ap, an, gp, gn, dp, dn, fd, w, b2, a2):
    return pl.pallas_call(
        _tc_final_body,
        out_shape=(
            jax.ShapeDtypeStruct((_N_D, _D), jnp.float32),
            jax.ShapeDtypeStruct((1, 1), jnp.float32),
        ),
    )(ap, an, gp, gn, dp, dn, fd, w, b2, a2)


def kernel(feat_src, feat_dst, edge_index, W, b, prelu_a, fc_W, fc_b, attn):
    del fc_W, fc_b, attn
    src = edge_index[0]
    dst = edge_index[1]
    b2 = b.reshape(1, _D)
    a2 = prelu_a.reshape(1, 1)

    table = _tc_norm(feat_src, W, b2, a2)

    pad_p = 16 * _CP * _B - _E
    i32 = src.dtype
    srcp = jnp.concatenate([src, jnp.zeros((pad_p,), i32)]).reshape(16, _CP, _B)
    dstp = jnp.concatenate([dst, jnp.full((pad_p,), _TRASH, i32)]).reshape(16, _CP, _B)
    eidx = jnp.arange(_E, dtype=i32)
    dm = jnp.where(eidx % 2 == 0, dst, _N_D + (eidx % (_RP - _N_D)))
    dstm = jnp.concatenate([dm, jnp.full((pad_p,), _TRASH, i32)]).reshape(16, _CP, _B)
    srcp2 = jnp.stack([srcp, srcp + _N_S])

    z128 = jnp.zeros((_B, _D), jnp.float32)
    z1 = jnp.zeros((_B,), jnp.float32)
    ones = jnp.ones((_B,), jnp.float32)

    sump, sumn, dp, dn = _sc_segsum(
        srcp2, dstp, dstm, table, z128, z1, ones)

    z, loss = _tc_final(
        sump[0, :_N_D], sumn[0, :_N_D], sump[1, :_N_D], sumn[1, :_N_D],
        dp[:_N_D, None], dn[:_N_D, None], feat_dst, W, b2, a2)
    return loss[0, 0], z
